# baseline (speedup 1.0000x reference)
import jax
import jax.numpy as jnp
from jax.experimental import pallas as pl

_NE = 8
_K = 2


def _router_kernel(h_ref, w_ref, probs_ref, idx_ref):
    h = h_ref[...]
    w = w_ref[...]
    logits = jax.lax.dot_general(
        h, w, (((1,), (1,)), ((), ())), preferred_element_type=jnp.float32
    )
    iota = jax.lax.broadcasted_iota(jnp.int32, logits.shape, 1)
    m1 = jnp.max(logits, axis=-1, keepdims=True)
    i1 = jnp.min(jnp.where(logits == m1, iota, _NE), axis=-1, keepdims=True)
    neg_inf = jnp.float32(-jnp.inf)
    masked = jnp.where(iota == i1, neg_inf, logits)
    m2 = jnp.max(masked, axis=-1, keepdims=True)
    i2 = jnp.min(jnp.where(masked == m2, iota, _NE), axis=-1, keepdims=True)
    t = jnp.exp(m2 - m1)
    denom = 1.0 + t
    p1 = 1.0 / denom
    p2 = t / denom
    probs_ref[...] = jnp.concatenate([p1, p2], axis=-1)
    idx_ref[...] = jnp.concatenate([i1, i2], axis=-1)


@jax.jit
def kernel(hidden_states, weight):
    h = hidden_states.reshape(-1, hidden_states.shape[-1])
    n, hd = h.shape
    blk = 1024
    probs, idx = pl.pallas_call(
        _router_kernel,
        grid=(n // blk,),
        in_specs=[
            pl.BlockSpec((blk, hd), lambda i: (i, 0)),
            pl.BlockSpec((_NE, hd), lambda i: (0, 0)),
        ],
        out_specs=[
            pl.BlockSpec((blk, _K), lambda i: (i, 0)),
            pl.BlockSpec((blk, _K), lambda i: (i, 0)),
        ],
        out_shape=[
            jax.ShapeDtypeStruct((n, _K), jnp.float32),
            jax.ShapeDtypeStruct((n, _K), jnp.int32),
        ],
    )(h, weight)
    return probs, idx
